# baseline (speedup 1.0000x reference)
import functools

import jax
import jax.numpy as jnp
from jax.experimental import pallas as pl
from jax.experimental.pallas import tpu as pltpu

_B, _T, _E, _NE = 4, 2048, 2048, 64
_M = _B * _T
_BLK = 512
_KMAX = 8


def _router_kernel(k_ref, x_ref, wh_ref, wl_ref, z_ref, o_ref):
    k = k_ref[0]
    xb = x_ref[...]
    x_hi = xb.astype(jnp.bfloat16)
    x_lo = (xb - x_hi.astype(jnp.float32)).astype(jnp.bfloat16)
    h2 = jnp.dot(x_hi, wh_ref[...], preferred_element_type=jnp.float32)
    h2 += jnp.dot(x_hi, wl_ref[...], preferred_element_type=jnp.float32)
    h2 += jnp.dot(x_lo, wh_ref[...], preferred_element_type=jnp.float32)
    prelim = h2[:, :_NE]
    noise = h2[:, _NE:]
    sp = jnp.maximum(noise, 0.0) + jnp.log1p(jnp.exp(-jnp.abs(noise)))
    h = prelim + z_ref[...] + sp

    work = h
    removed = jnp.zeros((h.shape[0], 1), jnp.int32)
    done = jnp.zeros((h.shape[0], 1), jnp.bool_)
    thr = jnp.full((h.shape[0], 1), -jnp.inf, jnp.float32)
    row_max = jnp.max(h, axis=1, keepdims=True)
    for _ in range(_KMAX):
        m = jnp.max(work, axis=1, keepdims=True)
        eq = work == m
        c = jnp.sum(eq.astype(jnp.int32), axis=1, keepdims=True)
        thr = jnp.where(done, thr, m)
        done = jnp.logical_or(done, removed + c >= k)
        removed = removed + c
        work = jnp.where(eq, -jnp.inf, work)

    keep = h >= thr
    e = jnp.where(keep, jnp.exp(h - row_max), 0.0)
    o_ref[...] = e / jnp.sum(e, axis=1, keepdims=True)


def kernel(x, k, W_g, W_noise):
    xm = x.reshape(_M, _E)
    w = jnp.concatenate([W_g, W_noise], axis=1)
    w_hi = w.astype(jnp.bfloat16)
    w_lo = (w - w_hi.astype(jnp.float32)).astype(jnp.bfloat16)
    z = jax.random.normal(jax.random.key(42), (_B, _T, _NE), dtype=jnp.float32)
    zm = z.reshape(_M, _NE)
    ks = jnp.asarray(k, jnp.int32).reshape(1)

    out = pl.pallas_call(
        _router_kernel,
        grid=(_M // _BLK,),
        in_specs=[
            pl.BlockSpec(memory_space=pltpu.SMEM),
            pl.BlockSpec((_BLK, _E), lambda i: (i, 0)),
            pl.BlockSpec((_E, 2 * _NE), lambda i: (0, 0)),
            pl.BlockSpec((_E, 2 * _NE), lambda i: (0, 0)),
            pl.BlockSpec((_BLK, _NE), lambda i: (i, 0)),
        ],
        out_specs=pl.BlockSpec((_BLK, _NE), lambda i: (i, 0)),
        out_shape=jax.ShapeDtypeStruct((_M, _NE), jnp.float32),
    )(ks, xm, w_hi, w_lo, zm)
    return out.reshape(_B, _T, _NE)


# speedup vs baseline: 1.9500x; 1.9500x over previous
import jax
import jax.numpy as jnp
from jax.experimental import pallas as pl
from jax.experimental.pallas import tpu as pltpu

_B, _T, _E, _NE = 4, 2048, 2048, 64
_M = _B * _T
_BLK = 512
_KMAX = 8


def _router_kernel(k_ref, x_ref, w_ref, zt_ref, o_ref):
    k = k_ref[0]
    h2 = jnp.dot(x_ref[...], w_ref[...], preferred_element_type=jnp.float32)
    h2t = h2.T
    prelim = h2t[:_NE, :]
    noise = h2t[_NE:, :]
    sp = jnp.maximum(noise, 0.0) + jnp.log1p(jnp.exp(-jnp.abs(noise)))
    h = prelim + zt_ref[...] + sp

    work = h
    removed = jnp.zeros((1, h.shape[1]), jnp.int32)
    done = jnp.zeros((1, h.shape[1]), jnp.bool_)
    thr = jnp.full((1, h.shape[1]), -jnp.inf, jnp.float32)
    col_max = jnp.max(h, axis=0, keepdims=True)
    for _ in range(_KMAX):
        m = jnp.max(work, axis=0, keepdims=True)
        eq = work == m
        c = jnp.sum(eq.astype(jnp.int32), axis=0, keepdims=True)
        thr = jnp.where(done, thr, m)
        done = jnp.logical_or(done, removed + c >= k)
        removed = removed + c
        work = jnp.where(eq, -jnp.inf, work)

    keep = h >= thr
    e = jnp.where(keep, jnp.exp(h - col_max), 0.0)
    ot = e / jnp.sum(e, axis=0, keepdims=True)
    o_ref[...] = ot.T


def kernel(x, k, W_g, W_noise):
    xm = x.reshape(_M, _E)
    w = jnp.concatenate([W_g, W_noise], axis=1)
    z = jax.random.normal(jax.random.key(42), (_B, _T, _NE), dtype=jnp.float32)
    zt = z.reshape(_M, _NE).T
    ks = jnp.asarray(k, jnp.int32).reshape(1)

    out = pl.pallas_call(
        _router_kernel,
        grid=(_M // _BLK,),
        in_specs=[
            pl.BlockSpec(memory_space=pltpu.SMEM),
            pl.BlockSpec((_BLK, _E), lambda i: (i, 0)),
            pl.BlockSpec((_E, 2 * _NE), lambda i: (0, 0)),
            pl.BlockSpec((_NE, _BLK), lambda i: (0, i)),
        ],
        out_specs=pl.BlockSpec((_BLK, _NE), lambda i: (i, 0)),
        out_shape=jax.ShapeDtypeStruct((_M, _NE), jnp.float32),
    )(ks, xm, w, zt)
    return out.reshape(_B, _T, _NE)


# speedup vs baseline: 2.0825x; 1.0679x over previous
import jax
import jax.numpy as jnp
from jax.experimental import pallas as pl
from jax.experimental.pallas import tpu as pltpu

_B, _T, _E, _NE = 4, 2048, 2048, 64
_M = _B * _T
_BLK = 1024
_KMAX = 8


def _router_kernel(k_ref, x_ref, w_ref, zt_ref, o_ref):
    k = k_ref[0]
    h2 = jnp.dot(x_ref[...], w_ref[...], preferred_element_type=jnp.float32)
    h2t = h2.T
    prelim = h2t[:_NE, :]
    noise = h2t[_NE:, :]
    sp = jnp.maximum(noise, 0.0) + jnp.log1p(jnp.exp(-jnp.abs(noise)))
    h = prelim + zt_ref[...] + sp

    work = h
    removed = jnp.zeros((1, h.shape[1]), jnp.int32)
    done = jnp.zeros((1, h.shape[1]), jnp.bool_)
    thr = jnp.full((1, h.shape[1]), -jnp.inf, jnp.float32)
    col_max = jnp.max(h, axis=0, keepdims=True)
    for _ in range(_KMAX):
        m = jnp.max(work, axis=0, keepdims=True)
        eq = work == m
        c = jnp.sum(eq.astype(jnp.int32), axis=0, keepdims=True)
        thr = jnp.where(done, thr, m)
        done = jnp.logical_or(done, removed + c >= k)
        removed = removed + c
        work = jnp.where(eq, -jnp.inf, work)

    keep = h >= thr
    e = jnp.where(keep, jnp.exp(h - col_max), 0.0)
    ot = e / jnp.sum(e, axis=0, keepdims=True)
    o_ref[...] = ot.T


def kernel(x, k, W_g, W_noise):
    xm = x.reshape(_M, _E)
    w = jnp.concatenate([W_g, W_noise], axis=1)
    z = jax.random.normal(jax.random.key(42), (_B, _T, _NE), dtype=jnp.float32)
    zt = z.reshape(_M, _NE).T
    ks = jnp.asarray(k, jnp.int32).reshape(1)

    out = pl.pallas_call(
        _router_kernel,
        grid=(_M // _BLK,),
        in_specs=[
            pl.BlockSpec(memory_space=pltpu.SMEM),
            pl.BlockSpec((_BLK, _E), lambda i: (i, 0)),
            pl.BlockSpec((_E, 2 * _NE), lambda i: (0, 0)),
            pl.BlockSpec((_NE, _BLK), lambda i: (0, i)),
        ],
        out_specs=pl.BlockSpec((_BLK, _NE), lambda i: (i, 0)),
        out_shape=jax.ShapeDtypeStruct((_M, _NE), jnp.float32),
    )(ks, xm, w, zt)
    return out.reshape(_B, _T, _NE)
